# baseline (speedup 1.0000x reference)
import jax
import jax.numpy as jnp
from jax.experimental import pallas as pl
from jax.experimental.pallas import tpu as pltpu

_TT = 64
_NK = 2


def _body(x_ref, w_ref, ct_ref, tt_ref, o_ref, xt_ref):
    ch, tt_len, in_dim = x_ref.shape[1:]
    dk = w_ref.shape[1]
    k = pl.program_id(2)

    @pl.when(k == 0)
    def _():
        xb = x_ref[0].astype(jnp.bfloat16)
        xt_ref[...] = jnp.swapaxes(xb, 0, 1).reshape(tt_len * ch, in_dim)

    acc = jax.lax.dot_general(
        xt_ref[...], w_ref[...],
        (((1,), (0,)), ((), ())),
        preferred_element_type=jnp.float32,
    ).reshape(tt_len, ch, dk)
    o_ref[...] = acc + tt_ref[...][:, None, :] + ct_ref[...][None, :, :]


def kernel(x, W, bias, chan_table, time_table, alpha_c, alpha_t):
    b, ch, t_len, in_dim = x.shape
    d = W.shape[0]
    dk = d // _NK
    wt = W.T.astype(jnp.bfloat16)
    ct = (alpha_c * chan_table).astype(jnp.bfloat16)
    tvec = (bias[None, :] + alpha_t * time_table).astype(jnp.bfloat16)
    n_t = t_len // _TT

    out = pl.pallas_call(
        _body,
        grid=(b, n_t, _NK),
        in_specs=[
            pl.BlockSpec((1, ch, _TT, in_dim), lambda i, j, k: (i, 0, j, 0)),
            pl.BlockSpec((in_dim, dk), lambda i, j, k: (0, k)),
            pl.BlockSpec((ch, dk), lambda i, j, k: (0, k)),
            pl.BlockSpec((_TT, dk), lambda i, j, k: (j, k)),
        ],
        out_specs=pl.BlockSpec((_TT, ch, dk),
                               lambda i, j, k: (i * n_t + j, 0, k)),
        out_shape=jax.ShapeDtypeStruct((b * t_len, ch, d), jnp.float32),
        scratch_shapes=[pltpu.VMEM((_TT * ch, in_dim), jnp.bfloat16)],
        compiler_params=pltpu.CompilerParams(
            dimension_semantics=("parallel", "parallel", "arbitrary"),
        ),
    )(x, wt, ct, tvec)
    return out.reshape(b, t_len * ch, d)


# speedup vs baseline: 1.4117x; 1.4117x over previous
import jax
import jax.numpy as jnp
from jax.experimental import pallas as pl
from jax.experimental.pallas import tpu as pltpu

_TT = 64


def _body(x_ref, w_ref, ct_ref, tt_ref, o_ref):
    ch, tt_len, in_dim = x_ref.shape[1:]
    d = w_ref.shape[1]
    xb = x_ref[0].astype(jnp.bfloat16)
    xt = jnp.swapaxes(xb, 0, 1)
    acc = jax.lax.dot_general(
        xt.reshape(tt_len * ch, in_dim), w_ref[...],
        (((1,), (0,)), ((), ())),
        preferred_element_type=jnp.float32,
    ).reshape(tt_len, ch, d)
    o_ref[...] = acc + tt_ref[...][:, None, :] + ct_ref[...][None, :, :]


def kernel(x, W, bias, chan_table, time_table, alpha_c, alpha_t):
    b, ch, t_len, in_dim = x.shape
    d = W.shape[0]
    wt = W.T.astype(jnp.bfloat16)
    ct = (alpha_c * chan_table).astype(jnp.bfloat16)
    tvec = (bias[None, :] + alpha_t * time_table).astype(jnp.bfloat16)
    n_t = t_len // _TT

    out = pl.pallas_call(
        _body,
        grid=(b, n_t),
        in_specs=[
            pl.BlockSpec((1, ch, _TT, in_dim), lambda i, j: (i, 0, j, 0)),
            pl.BlockSpec((in_dim, d), lambda i, j: (0, 0)),
            pl.BlockSpec((ch, d), lambda i, j: (0, 0)),
            pl.BlockSpec((_TT, d), lambda i, j: (j, 0)),
        ],
        out_specs=pl.BlockSpec((_TT, ch, d), lambda i, j: (i * n_t + j, 0, 0)),
        out_shape=jax.ShapeDtypeStruct((b * t_len, ch, d), jnp.float32),
        compiler_params=pltpu.CompilerParams(
            dimension_semantics=("arbitrary", "arbitrary"),
        ),
    )(x, wt, ct, tvec)
    return out.reshape(b, t_len * ch, d)
